# baseline (speedup 1.0000x reference)
import functools

import jax
import jax.numpy as jnp
from jax import lax
from jax.experimental import pallas as pl
from jax.experimental.pallas import tpu as pltpu
from jax.experimental.pallas import tpu_sc as plsc

BATCH = 16384
FIELDS = 26
N = BATCH * FIELDS
EMB = 64
NC = 2
NS = 16
NW = NC * NS
TPW = N // NW
BLK = 128
NB = TPW // BLK
L = 16


def _pt_matmul(lhs, rhs):
    def body(l_ref, r_ref, o_ref):
        o_ref[...] = jnp.dot(l_ref[...], r_ref[...],
                             preferred_element_type=jnp.float32)
    return pl.pallas_call(
        body,
        out_shape=jax.ShapeDtypeStruct((100, 12800), jnp.float32),
    )(lhs, rhs)


def _splat(v):
    return jnp.full((L,), v, dtype=jnp.int32)


_mesh = plsc.VectorSubcoreMesh(core_axis_name="c", subcore_axis_name="s")


@functools.partial(
    pl.kernel,
    mesh=_mesh,
    compiler_params=pltpu.CompilerParams(needs_layout_passes=False),
    out_type=jax.ShapeDtypeStruct((N * EMB,), jnp.float32),
    scratch_types=[
        pltpu.VMEM((TPW,), jnp.int32),
        pltpu.VMEM((3304,), jnp.float32),
        pltpu.VMEM((BLK,), jnp.int32),
        pltpu.VMEM((BLK,), jnp.int32),
        pltpu.VMEM((BLK,), jnp.int32),
        pltpu.VMEM((BLK,), jnp.int32),
        pltpu.VMEM((BLK, 128), jnp.float32),
        pltpu.VMEM((BLK, 128), jnp.float32),
        pltpu.VMEM((BLK * EMB,), jnp.float32),
        pltpu.VMEM((BLK * EMB,), jnp.float32),
        pltpu.SemaphoreType.DMA,
        pltpu.SemaphoreType.DMA,
        pltpu.SemaphoreType.DMA,
        pltpu.SemaphoreType.DMA,
    ],
)
def _sc_lookup(idx_hbm, pt_hbm, ct_hbm, out_hbm,
               idx_v, ct_v, gidx0, gidx1, i30, i31, rows0, rows1,
               out0, out1, gsem0, gsem1, osem0, osem1):
    gidxs = (gidx0, gidx1)
    i3s = (i30, i31)
    rows = (rows0, rows1)
    outs = (out0, out1)
    gsems = (gsem0, gsem1)
    osems = (osem0, osem1)
    wid = lax.axis_index("s") * NC + lax.axis_index("c")
    base = wid * TPW

    pltpu.sync_copy(idx_hbm.at[pl.ds(base, TPW)], idx_v)
    pltpu.sync_copy(ct_hbm, ct_v)

    def make_idx(bb, slot):
        for v in range(BLK // L):
            iv = idx_v[pl.ds(bb * BLK + v * L, L)]
            f = iv.astype(jnp.float32) + 0.5
            i12 = (f * 0.01).astype(jnp.int32)
            i3 = iv - i12 * 100
            gidxs[slot][pl.ds(v * L, L)] = i12
            i3s[slot][pl.ds(v * L, L)] = i3

    def fire(slot):
        pltpu.async_copy(pt_hbm.at[gidxs[slot]], rows[slot], gsems[slot])

    def drain(slot):
        pltpu.make_async_copy(pt_hbm.at[gidxs[slot]], rows[slot],
                              gsems[slot]).wait()

    def owait(slot):
        pltpu.make_async_copy(outs[slot],
                              out_hbm.at[pl.ds(base * EMB, BLK * EMB)],
                              osems[slot]).wait()

    def compute(slot):
        iota = lax.iota(jnp.int32, L)
        jjadd = iota >> 3
        r2rot = [(iota + r) & 7 for r in range(8)]
        j3rot = [(iota + j) & 3 for j in range(4)]

        @plsc.parallel_loop(0, BLK // L, 1, unroll=2)
        def gbody(g):
            tvec = iota + g * L
            tvec64 = tvec * EMB
            i3g = i3s[slot][pl.ds(g * L, L)]
            cbase = i3g * 33
            crr = [cbase + (r2rot[r] << 2) for r in range(8)]
            cv = [[plsc.load_gather(ct_v, [crr[r] + j3rot[j]])
                   for j in range(4)] for r in range(8)]
            for jj in range(16):
                jjl = (jjadd + jj) & 15
                jj8 = jjl << 3
                jj4 = tvec64 + (jjl << 2)
                pv = [plsc.load_gather(rows[slot], [tvec, jj8 + r2rot[r]])
                      for r in range(8)]
                for j3 in range(4):
                    acc = pv[0] * cv[0][j3]
                    for r in range(1, 8):
                        acc = acc + pv[r] * cv[r][j3]
                    plsc.store_scatter(outs[slot], [jj4 + j3rot[j3]], acc)

    make_idx(0, 0)
    fire(0)

    def outer(i, carry):
        for par in range(2):
            bb = 2 * i + par

            @pl.when(bb + 1 < NB)
            def _prefetch():
                make_idx(bb + 1, 1 - par)
                fire(1 - par)

            drain(par)

            @pl.when(bb >= 2)
            def _wait_out():
                owait(par)

            compute(par)
            pltpu.async_copy(
                outs[par],
                out_hbm.at[pl.ds((base + bb * BLK) * EMB, BLK * EMB)],
                osems[par])
        return carry

    lax.fori_loop(0, NB // 2, outer, 0)
    owait(0)
    owait(1)


def kernel(indices, core0, core1, core2):
    lhs = core0[0].reshape(100, 32)
    eye4 = jnp.eye(4, dtype=core1.dtype)
    rhs = (eye4[:, None, None, :, None, None]
           * core1[None, :, :, None, :, :]).reshape(32, 12800)
    pt = _pt_matmul(lhs, rhs).reshape(10000, 128)
    ct = jnp.transpose(core2[:, :, :, 0], (1, 0, 2)).reshape(100, 32)
    ct = jnp.pad(ct, ((0, 0), (0, 1))).reshape(3300)
    ct = jnp.pad(ct, (0, 4))
    idx = indices.reshape(-1)
    out = _sc_lookup(idx, pt, ct)
    return out.reshape(BATCH, FIELDS, EMB)


# speedup vs baseline: 1.0337x; 1.0337x over previous
import functools

import jax
import jax.numpy as jnp
from jax import lax
from jax.experimental import pallas as pl
from jax.experimental.pallas import tpu as pltpu
from jax.experimental.pallas import tpu_sc as plsc

BATCH = 16384
FIELDS = 26
N = BATCH * FIELDS
EMB = 64
NC = 2
NS = 16
NW = NC * NS
TPW = N // NW
BLK = 128
NB = TPW // BLK
L = 16


def _pt_matmul(lhs, rhs):
    def body(l_ref, r_ref, o_ref):
        o_ref[...] = jnp.dot(l_ref[...], r_ref[...],
                             preferred_element_type=jnp.float32)
    return pl.pallas_call(
        body,
        out_shape=jax.ShapeDtypeStruct((100, 12800), jnp.float32),
    )(lhs, rhs)


def _splat(v):
    return jnp.full((L,), v, dtype=jnp.int32)


_mesh = plsc.VectorSubcoreMesh(core_axis_name="c", subcore_axis_name="s")


@functools.partial(
    pl.kernel,
    mesh=_mesh,
    compiler_params=pltpu.CompilerParams(needs_layout_passes=False),
    out_type=jax.ShapeDtypeStruct((N * EMB,), jnp.float32),
    scratch_types=[
        pltpu.VMEM((TPW,), jnp.int32),
        pltpu.VMEM((1704,), jnp.float32),
        pltpu.VMEM((BLK,), jnp.int32),
        pltpu.VMEM((BLK,), jnp.int32),
        pltpu.VMEM((BLK,), jnp.int32),
        pltpu.VMEM((BLK,), jnp.int32),
        pltpu.VMEM((BLK, 128), jnp.float32),
        pltpu.VMEM((BLK, 128), jnp.float32),
        pltpu.VMEM((BLK * EMB,), jnp.float32),
        pltpu.VMEM((BLK * EMB,), jnp.float32),
        pltpu.SemaphoreType.DMA,
        pltpu.SemaphoreType.DMA,
        pltpu.SemaphoreType.DMA,
        pltpu.SemaphoreType.DMA,
    ],
)
def _sc_lookup(idx_hbm, pt_hbm, ct_hbm, out_hbm,
               idx_v, ct_v, gidx0, gidx1, i30, i31, rows0, rows1,
               out0, out1, gsem0, gsem1, osem0, osem1):
    gidxs = (gidx0, gidx1)
    i3s = (i30, i31)
    rows = (rows0, rows1)
    outs = (out0, out1)
    gsems = (gsem0, gsem1)
    osems = (osem0, osem1)
    wid = lax.axis_index("s") * NC + lax.axis_index("c")
    base = wid * TPW

    pltpu.sync_copy(idx_hbm.at[pl.ds(base, TPW)], idx_v)
    pltpu.sync_copy(ct_hbm, ct_v)

    def make_idx(bb, slot):
        for v in range(BLK // L):
            iv = idx_v[pl.ds(bb * BLK + v * L, L)]
            f = iv.astype(jnp.float32) + 0.5
            i12 = (f * 0.01).astype(jnp.int32)
            i3 = iv - i12 * 100
            gidxs[slot][pl.ds(v * L, L)] = i12
            i3s[slot][pl.ds(v * L, L)] = i3

    def fire(slot):
        pltpu.async_copy(pt_hbm.at[gidxs[slot]], rows[slot], gsems[slot])

    def drain(slot):
        pltpu.make_async_copy(pt_hbm.at[gidxs[slot]], rows[slot],
                              gsems[slot]).wait()

    def owait(slot):
        pltpu.make_async_copy(outs[slot],
                              out_hbm.at[pl.ds(base * EMB, BLK * EMB)],
                              osems[slot]).wait()

    def compute(slot):
        iota = lax.iota(jnp.int32, L)
        jjadd = iota >> 2
        krot = [(iota + k) & 3 for k in range(4)]
        j3rot = [(iota + j) & 3 for j in range(4)]

        def gbody(g, carry):
            tvec = iota + g * L
            tvec64 = tvec * EMB
            i3g = i3s[slot][pl.ds(g * L, L)]
            cbase = i3g * 17
            ckb = [cbase + (krot[k] << 2) for k in range(4)]
            cw = [[plsc.bitcast(plsc.load_gather(ct_v, [ckb[k] + j3rot[j]]),
                                jnp.bfloat16)
                   for j in range(4)] for k in range(4)]
            sidx = [tvec64 + j3rot[j] for j in range(4)]
            for jj in range(16):
                jjl = (jjadd + jj) & 15
                jj4 = jjl << 2
                pw = [plsc.bitcast(plsc.load_gather(rows[slot],
                                                    [tvec, jj4 + krot[k]]),
                                   jnp.bfloat16)
                      for k in range(4)]
                for j3 in range(4):
                    acc2 = pw[0] * cw[0][j3]
                    for k in range(1, 4):
                        acc2 = acc2 + pw[k] * cw[k][j3]
                    lo, hi = plsc.unpack(acc2,
                                         format=plsc.PackFormat.INTERLEAVED)
                    plsc.store_scatter(outs[slot], [sidx[j3] + jj4], lo + hi)
            return carry
        lax.fori_loop(0, BLK // L, gbody, 0)

    make_idx(0, 0)
    fire(0)

    def outer(i, carry):
        for par in range(2):
            bb = 2 * i + par

            @pl.when(bb + 1 < NB)
            def _prefetch():
                make_idx(bb + 1, 1 - par)
                fire(1 - par)

            drain(par)

            @pl.when(bb >= 2)
            def _wait_out():
                owait(par)

            compute(par)
            pltpu.async_copy(
                outs[par],
                out_hbm.at[pl.ds((base + bb * BLK) * EMB, BLK * EMB)],
                osems[par])
        return carry

    lax.fori_loop(0, NB // 2, outer, 0)
    owait(0)
    owait(1)


def kernel(indices, core0, core1, core2):
    lhs = core0[0].reshape(100, 32)
    eye4 = jnp.eye(4, dtype=core1.dtype)
    rhs = (eye4[:, None, None, :, None, None]
           * core1[None, :, :, None, :, :]).reshape(32, 12800)
    pt = _pt_matmul(lhs, rhs).reshape(10000, 64, 2)
    ptb = lax.bitcast_convert_type(pt.astype(jnp.bfloat16), jnp.uint16)
    ptw = ptb[..., 0].astype(jnp.uint32) | (ptb[..., 1].astype(jnp.uint32)
                                            << 16)
    ptf = lax.bitcast_convert_type(ptw, jnp.float32)
    pt = jnp.concatenate(
        [ptf, jnp.zeros((10000, 64), jnp.float32)], axis=1)
    g2 = core2[:, :, :, 0].reshape(4, 2, 100, 4)
    gb = lax.bitcast_convert_type(g2.astype(jnp.bfloat16), jnp.uint16)
    cw = gb[:, 0].astype(jnp.uint32) | (gb[:, 1].astype(jnp.uint32) << 16)
    ctf = lax.bitcast_convert_type(cw, jnp.float32)
    ct = jnp.transpose(ctf, (1, 0, 2)).reshape(100, 16)
    ct = jnp.pad(ct, ((0, 0), (0, 1))).reshape(1700)
    ct = jnp.pad(ct, (0, 4))
    idx = indices.reshape(-1)
    out = _sc_lookup(idx, pt, ct)
    return out.reshape(BATCH, FIELDS, EMB)


# speedup vs baseline: 1.8309x; 1.7712x over previous
import functools

import jax
import jax.numpy as jnp
from jax import lax
from jax.experimental import pallas as pl
from jax.experimental.pallas import tpu as pltpu
from jax.experimental.pallas import tpu_sc as plsc

BATCH = 16384
FIELDS = 26
N = BATCH * FIELDS
EMB = 64
NC = 2
NS = 16
NW = NC * NS
TPW = N // NW
BLK = 128
NB = TPW // BLK
L = 16


def _pt_matmul(lhs, rhs):
    def body(l_ref, r_ref, o_ref):
        o_ref[...] = jnp.dot(l_ref[...], r_ref[...],
                             preferred_element_type=jnp.float32)
    return pl.pallas_call(
        body,
        out_shape=jax.ShapeDtypeStruct((100, 12800), jnp.float32),
    )(lhs, rhs)


def _splat(v):
    return jnp.full((L,), v, dtype=jnp.int32)


_mesh = plsc.VectorSubcoreMesh(core_axis_name="c", subcore_axis_name="s")


@functools.partial(
    pl.kernel,
    mesh=_mesh,
    compiler_params=pltpu.CompilerParams(needs_layout_passes=False),
    out_type=jax.ShapeDtypeStruct((N * EMB,), jnp.float32),
    scratch_types=[
        pltpu.VMEM((TPW,), jnp.int32),
        pltpu.VMEM((3304,), jnp.float32),
        pltpu.VMEM((BLK,), jnp.int32),
        pltpu.VMEM((BLK,), jnp.int32),
        pltpu.VMEM((BLK,), jnp.int32),
        pltpu.VMEM((BLK,), jnp.int32),
        pltpu.VMEM((BLK, 128), jnp.float32),
        pltpu.VMEM((BLK, 128), jnp.float32),
        pltpu.VMEM((BLK * EMB,), jnp.float32),
        pltpu.VMEM((BLK * EMB,), jnp.float32),
        pltpu.SemaphoreType.DMA,
        pltpu.SemaphoreType.DMA,
        pltpu.SemaphoreType.DMA,
        pltpu.SemaphoreType.DMA,
    ],
)
def _sc_lookup(idx_hbm, pt_hbm, ct_hbm, out_hbm,
               idx_v, ct_v, gidx0, gidx1, i30, i31, rows0, rows1,
               out0, out1, gsem0, gsem1, osem0, osem1):
    gidxs = (gidx0, gidx1)
    i3s = (i30, i31)
    rows = (rows0, rows1)
    outs = (out0, out1)
    gsems = (gsem0, gsem1)
    osems = (osem0, osem1)
    wid = lax.axis_index("s") * NC + lax.axis_index("c")
    base = wid * TPW

    pltpu.sync_copy(idx_hbm.at[pl.ds(base, TPW)], idx_v)
    pltpu.sync_copy(ct_hbm, ct_v)

    def make_idx(bb, slot):
        for v in range(BLK // L):
            iv = idx_v[pl.ds(bb * BLK + v * L, L)]
            f = iv.astype(jnp.float32) + 0.5
            i12 = (f * 0.01).astype(jnp.int32)
            i3 = iv - i12 * 100
            gidxs[slot][pl.ds(v * L, L)] = i12
            i3s[slot][pl.ds(v * L, L)] = i3

    def fire(slot):
        pltpu.async_copy(pt_hbm.at[gidxs[slot]], rows[slot], gsems[slot])

    def drain(slot):
        pltpu.make_async_copy(pt_hbm.at[gidxs[slot]], rows[slot],
                              gsems[slot]).wait()

    def owait(slot):
        pltpu.make_async_copy(outs[slot],
                              out_hbm.at[pl.ds(base * EMB, BLK * EMB)],
                              osems[slot]).wait()

    def compute(slot):
        iota = lax.iota(jnp.int32, L)
        jjadd = iota >> 3
        r2rot = [(iota + r) & 7 for r in range(8)]
        j3rot = [(iota + j) & 3 for j in range(4)]

        def gbody(g, carry):
            tvec = iota + g * L
            tvec64 = tvec * EMB
            i3g = i3s[slot][pl.ds(g * L, L)]
            cbase = i3g * 33
            crr = [cbase + (r2rot[r] << 2) for r in range(8)]
            cv = [[plsc.load_gather(ct_v, [crr[r] + j3rot[j]])
                   for j in range(4)] for r in range(8)]
            for jj in range(16):
                jjl = (jjadd + jj) & 15
                jj8 = jjl << 3
                jj4 = tvec64 + (jjl << 2)
                pv = [plsc.load_gather(rows[slot], [tvec, jj8 + r2rot[r]])
                      for r in range(8)]
                for j3 in range(4):
                    acc_a = pv[0] * cv[0][j3]
                    acc_b = pv[1] * cv[1][j3]
                    for r in range(2, 8, 2):
                        acc_a = acc_a + pv[r] * cv[r][j3]
                        acc_b = acc_b + pv[r + 1] * cv[r + 1][j3]
                    plsc.store_scatter(outs[slot], [jj4 + j3rot[j3]],
                                       acc_a + acc_b)
            return carry
        lax.fori_loop(0, BLK // L, gbody, 0)

    make_idx(0, 0)
    fire(0)

    def outer(i, carry):
        for par in range(2):
            bb = 2 * i + par

            @pl.when(bb + 1 < NB)
            def _prefetch():
                make_idx(bb + 1, 1 - par)
                fire(1 - par)

            drain(par)

            @pl.when(bb >= 2)
            def _wait_out():
                owait(par)

            compute(par)
            pltpu.async_copy(
                outs[par],
                out_hbm.at[pl.ds((base + bb * BLK) * EMB, BLK * EMB)],
                osems[par])
        return carry

    lax.fori_loop(0, NB // 2, outer, 0)
    owait(0)
    owait(1)


def kernel(indices, core0, core1, core2):
    lhs = core0[0].reshape(100, 32)
    eye4 = jnp.eye(4, dtype=core1.dtype)
    rhs = (eye4[:, None, None, :, None, None]
           * core1[None, :, :, None, :, :]).reshape(32, 12800)
    pt = _pt_matmul(lhs, rhs).reshape(10000, 128)
    ct = jnp.transpose(core2[:, :, :, 0], (1, 0, 2)).reshape(100, 32)
    ct = jnp.pad(ct, ((0, 0), (0, 1))).reshape(3300)
    ct = jnp.pad(ct, (0, 4))
    idx = indices.reshape(-1)
    out = _sc_lookup(idx, pt, ct)
    return out.reshape(BATCH, FIELDS, EMB)


# speedup vs baseline: 1.9762x; 1.0794x over previous
import functools

import jax
import jax.numpy as jnp
from jax import lax
from jax.experimental import pallas as pl
from jax.experimental.pallas import tpu as pltpu
from jax.experimental.pallas import tpu_sc as plsc

BATCH = 16384
FIELDS = 26
N = BATCH * FIELDS
EMB = 64
NC = 2
NS = 16
NW = NC * NS
TPW = N // NW
BLK = 128
NB = TPW // BLK
L = 16


def _pt_matmul(lhs, rhs):
    def body(l_ref, r_ref, o_ref):
        o_ref[...] = jnp.dot(l_ref[...], r_ref[...],
                             preferred_element_type=jnp.float32)
    return pl.pallas_call(
        body,
        out_shape=jax.ShapeDtypeStruct((100, 12800), jnp.float32),
    )(lhs, rhs)


def _splat(v):
    return jnp.full((L,), v, dtype=jnp.int32)


_mesh = plsc.VectorSubcoreMesh(core_axis_name="c", subcore_axis_name="s")


@functools.partial(
    pl.kernel,
    mesh=_mesh,
    compiler_params=pltpu.CompilerParams(needs_layout_passes=False),
    out_type=jax.ShapeDtypeStruct((N * EMB,), jnp.float32),
    scratch_types=[
        pltpu.VMEM((TPW,), jnp.int32),
        pltpu.VMEM((3304,), jnp.float32),
        pltpu.VMEM((BLK,), jnp.int32),
        pltpu.VMEM((BLK,), jnp.int32),
        pltpu.VMEM((BLK,), jnp.int32),
        pltpu.VMEM((BLK,), jnp.int32),
        pltpu.VMEM((BLK, 128), jnp.float32),
        pltpu.VMEM((BLK, 128), jnp.float32),
        pltpu.VMEM((BLK * EMB,), jnp.float32),
        pltpu.VMEM((BLK * EMB,), jnp.float32),
        pltpu.SemaphoreType.DMA,
        pltpu.SemaphoreType.DMA,
        pltpu.SemaphoreType.DMA,
        pltpu.SemaphoreType.DMA,
    ],
)
def _sc_lookup(idx_hbm, pt_hbm, ct_hbm, out_hbm,
               idx_v, ct_v, gidx0, gidx1, i30, i31, rows0, rows1,
               out0, out1, gsem0, gsem1, osem0, osem1):
    gidxs = (gidx0, gidx1)
    i3s = (i30, i31)
    rows = (rows0, rows1)
    outs = (out0, out1)
    gsems = (gsem0, gsem1)
    osems = (osem0, osem1)
    wid = lax.axis_index("s") * NC + lax.axis_index("c")
    base = wid * TPW

    pltpu.sync_copy(idx_hbm.at[pl.ds(base, TPW)], idx_v)
    pltpu.sync_copy(ct_hbm, ct_v)

    def make_idx(bb, slot):
        for v in range(BLK // L):
            iv = idx_v[pl.ds(bb * BLK + v * L, L)]
            f = iv.astype(jnp.float32) + 0.5
            i12 = (f * 0.01).astype(jnp.int32)
            i3 = iv - i12 * 100
            gidxs[slot][pl.ds(v * L, L)] = i12
            i3s[slot][pl.ds(v * L, L)] = i3

    def fire(slot):
        pltpu.async_copy(pt_hbm.at[gidxs[slot]], rows[slot], gsems[slot])

    def drain(slot):
        pltpu.make_async_copy(pt_hbm.at[gidxs[slot]], rows[slot],
                              gsems[slot]).wait()

    def owait(slot):
        pltpu.make_async_copy(outs[slot],
                              out_hbm.at[pl.ds(base * EMB, BLK * EMB)],
                              osems[slot]).wait()

    def compute(slot):
        iota = lax.iota(jnp.int32, L)
        jjadd = iota >> 3
        r2rot = [(iota + r) & 7 for r in range(8)]
        j3rot = [(iota + j) & 3 for j in range(4)]

        def gbody(g, carry):
            tvec = iota + g * L
            tvec64 = tvec * EMB
            i3g = i3s[slot][pl.ds(g * L, L)]
            cbase = i3g * 33
            crr = [cbase + (r2rot[r] << 2) for r in range(8)]
            cv = [[plsc.load_gather(ct_v, [crr[r] + j3rot[j]])
                   for j in range(4)] for r in range(8)]
            for jj in range(16):
                jjl = (jjadd + jj) & 15
                jj8 = jjl << 3
                jj4 = tvec64 + (jjl << 2)
                pv = [plsc.load_gather(rows[slot], [tvec, jj8 + r2rot[r]])
                      for r in range(8)]
                for j3 in range(4):
                    s0 = pv[0] * cv[0][j3] + pv[4] * cv[4][j3]
                    s1 = pv[1] * cv[1][j3] + pv[5] * cv[5][j3]
                    s2 = pv[2] * cv[2][j3] + pv[6] * cv[6][j3]
                    s3 = pv[3] * cv[3][j3] + pv[7] * cv[7][j3]
                    plsc.store_scatter(outs[slot], [jj4 + j3rot[j3]],
                                       (s0 + s1) + (s2 + s3))
            return carry
        lax.fori_loop(0, BLK // L, gbody, 0)

    make_idx(0, 0)
    fire(0)

    def outer(i, carry):
        for par in range(2):
            bb = 2 * i + par

            @pl.when(bb + 1 < NB)
            def _prefetch():
                make_idx(bb + 1, 1 - par)
                fire(1 - par)

            drain(par)

            @pl.when(bb >= 2)
            def _wait_out():
                owait(par)

            compute(par)
            pltpu.async_copy(
                outs[par],
                out_hbm.at[pl.ds((base + bb * BLK) * EMB, BLK * EMB)],
                osems[par])
        return carry

    lax.fori_loop(0, NB // 2, outer, 0)
    owait(0)
    owait(1)


def kernel(indices, core0, core1, core2):
    lhs = core0[0].reshape(100, 32)
    eye4 = jnp.eye(4, dtype=core1.dtype)
    rhs = (eye4[:, None, None, :, None, None]
           * core1[None, :, :, None, :, :]).reshape(32, 12800)
    pt = _pt_matmul(lhs, rhs).reshape(10000, 128)
    ct = jnp.transpose(core2[:, :, :, 0], (1, 0, 2)).reshape(100, 32)
    ct = jnp.pad(ct, ((0, 0), (0, 1))).reshape(3300)
    ct = jnp.pad(ct, (0, 4))
    idx = indices.reshape(-1)
    out = _sc_lookup(idx, pt, ct)
    return out.reshape(BATCH, FIELDS, EMB)
